# baseline (speedup 1.0000x reference)
import functools

import jax
import jax.numpy as jnp
from jax import lax
from jax.experimental import pallas as pl
from jax.experimental.pallas import tpu as pltpu
from jax.experimental.pallas import tpu_sc as plsc

N = 10000
E = 160000
D = 128
S = 2
NS = 16
ROWS_MAIN = 624
ROWS_TAIL = N - NS * ROWS_MAIN
EDGES_PER_TILE = E // NS
CHUNK = 80
NCHUNK = EDGES_PER_TILE // CHUNK
NBUF = 4
ROUNDS = NCHUNK // NBUF

BT = 1000
NB = N // BT


def _sc_body(xs_hbm, sidx_hbm, didx_hbm, out_hbm,
             sidx_v, didx_v, acc_sh,
             r0, r1, r2, r3, g0, g1, g2, g3, s0, s1, s2, s3,
             d0, d1, d2, d3, e0, e1, e2, e3):
    c = lax.axis_index("c")
    s = lax.axis_index("s")
    rows = (r0, r1, r2, r3)
    gsem = (g0, g1, g2, g3)
    ssem = (s0, s1, s2, s3)
    dsem = (d0, d1, d2, d3)
    esem = (e0, e1, e2, e3)
    row0 = s * ROWS_MAIN


    pltpu.sync_copy(xs_hbm.at[pl.ds(c * N + row0, ROWS_MAIN)],
                    acc_sh.at[pl.ds(row0, ROWS_MAIN)])

    @pl.when(s == 0)
    def _():
        pltpu.sync_copy(xs_hbm.at[pl.ds(c * N + NS * ROWS_MAIN, ROWS_TAIL)],
                        acc_sh.at[pl.ds(NS * ROWS_MAIN, ROWS_TAIL)])

    plsc.subcore_barrier()

    def start_sidx(j, b):
        pltpu.async_copy(sidx_hbm.at[c, s, pl.ds(j, 1)], sidx_v.at[pl.ds(b, 1)],
                         esem[b])

    def wait_sidx(b):
        pltpu.make_async_copy(sidx_hbm.at[c, s, pl.ds(0, 1)],
                              sidx_v.at[pl.ds(b, 1)], esem[b]).wait()

    def start_didx(j, b):
        pltpu.async_copy(didx_hbm.at[s, pl.ds(j, 1)], didx_v.at[pl.ds(b, 1)],
                         dsem[b])

    def wait_didx(b):
        pltpu.make_async_copy(didx_hbm.at[s, pl.ds(0, 1)],
                              didx_v.at[pl.ds(b, 1)], dsem[b]).wait()

    def start_gather(b):
        pltpu.async_copy(xs_hbm.at[sidx_v.at[b]], rows[b], gsem[b])

    def wait_gather(b):
        pltpu.make_async_copy(xs_hbm.at[sidx_v.at[0]], rows[b], gsem[b]).wait()

    def start_scatter(b):
        pltpu.async_copy(rows[b], acc_sh.at[didx_v.at[b]], ssem[b], add=True)

    def wait_scatter(b):
        pltpu.make_async_copy(rows[b], acc_sh.at[didx_v.at[0]], ssem[b]).wait()

    for b in range(NBUF):
        start_sidx(b, b)
        start_didx(b, b)
    for b in range(NBUF):
        wait_sidx(b)
        start_gather(b)

    def round_body(t, carry):
        j0 = t * NBUF
        for b in range(NBUF):
            wait_gather(b)
            wait_didx(b)
            start_scatter(b)
            start_sidx(j0 + NBUF + b, b)
        for b in range(NBUF):
            wait_scatter(b)
            start_didx(j0 + NBUF + b, b)
        for b in range(NBUF):
            wait_sidx(b)
            start_gather(b)
        return carry

    lax.fori_loop(0, ROUNDS - 1, round_body, 0)

    for b in range(NBUF):
        wait_gather(b)
        wait_didx(b)
        start_scatter(b)
    for b in range(NBUF):
        wait_scatter(b)

    for j in range(ROUNDS * NBUF, NCHUNK):
        start_sidx(j, 0)
        start_didx(j, 0)
        wait_sidx(0)
        wait_didx(0)
        start_gather(0)
        wait_gather(0)
        start_scatter(0)
        wait_scatter(0)

    plsc.subcore_barrier()
    pltpu.sync_copy(acc_sh.at[pl.ds(row0, ROWS_MAIN)],
                    out_hbm.at[pl.ds(c * N + row0, ROWS_MAIN)])

    @pl.when(s == 0)
    def _():
        pltpu.sync_copy(acc_sh.at[pl.ds(NS * ROWS_MAIN, ROWS_TAIL)],
                        out_hbm.at[pl.ds(c * N + NS * ROWS_MAIN, ROWS_TAIL)])


@functools.cache
def _sc_segsum():
    return pl.kernel(
        _sc_body,
        mesh=plsc.VectorSubcoreMesh(core_axis_name="c", subcore_axis_name="s"),
        out_type=jax.ShapeDtypeStruct((S * N, D), jnp.float32),
        scratch_types=[
            pltpu.VMEM((NBUF, CHUNK), jnp.int32),
            pltpu.VMEM((NBUF, CHUNK), jnp.int32),
            pltpu.VMEM_SHARED((N, D), jnp.float32),
        ] + [pltpu.VMEM((CHUNK, D), jnp.float32)] * NBUF
          + [pltpu.SemaphoreType.DMA] * (4 * NBUF),
    )


def _stats_body(x0, x1, g0, g1, w1s, w1, pv, out):
    i = pl.program_id(0)

    @pl.when(i == 0)
    def _():
        out[...] = jnp.zeros_like(out)

    e1 = pv[8:9, :]
    e2 = pv[9:10, :]
    h0 = g0[...] + e1 * x0[...]
    h1 = g1[...] + e1 * x1[...]
    hm = 0.5 * (g0[...] + g1[...]) + 0.5 * e2 * (x0[...] + x1[...])
    a0 = jnp.dot(h0, w1s[...], preferred_element_type=jnp.float32) + pv[0:1, :]
    a1 = jnp.dot(h1, w1s[...], preferred_element_type=jnp.float32) + pv[0:1, :]
    am = jnp.dot(hm, w1[...], preferred_element_type=jnp.float32) + pv[4:5, :]
    s_s = jnp.sum(a0, axis=0) + jnp.sum(a1, axis=0)
    q_s = jnp.sum(a0 * a0, axis=0) + jnp.sum(a1 * a1, axis=0)
    s_a = jnp.sum(am, axis=0)
    q_a = jnp.sum(am * am, axis=0)
    z = jnp.zeros_like(s_s)
    upd = jnp.stack([s_s, q_s, s_a, q_a, z, z, z, z])
    out[...] = out[...] + upd


def _apply_body(x0, x1, g0, g1, st, w1s, w2s, w1, w2, pv, out):
    e1 = pv[8:9, :]
    e2 = pv[9:10, :]
    h0 = g0[...] + e1 * x0[...]
    h1 = g1[...] + e1 * x1[...]
    hm = 0.5 * (g0[...] + g1[...]) + 0.5 * e2 * (x0[...] + x1[...])

    inv_ms = 1.0 / float(S * N)
    inv_ma = 1.0 / float(N)
    mean_s = st[0:1, :] * inv_ms
    var_s = st[1:2, :] * inv_ms - mean_s * mean_s
    scale_s = pv[2:3, :] * lax.rsqrt(var_s + 1e-5)
    shift_s = pv[3:4, :] - mean_s * scale_s
    mean_a = st[2:3, :] * inv_ma
    var_a = st[3:4, :] * inv_ma - mean_a * mean_a
    scale_a = pv[6:7, :] * lax.rsqrt(var_a + 1e-5)
    shift_a = pv[7:8, :] - mean_a * scale_a

    a0 = jnp.dot(h0, w1s[...], preferred_element_type=jnp.float32) + pv[0:1, :]
    a1 = jnp.dot(h1, w1s[...], preferred_element_type=jnp.float32) + pv[0:1, :]
    am = jnp.dot(hm, w1[...], preferred_element_type=jnp.float32) + pv[4:5, :]
    r0 = jnp.maximum(a0 * scale_s + shift_s, 0.0)
    r1 = jnp.maximum(a1 * scale_s + shift_s, 0.0)
    rm = jnp.maximum(am * scale_a + shift_a, 0.0)
    z0 = jnp.dot(r0, w2s[...], preferred_element_type=jnp.float32) + pv[1:2, :]
    z1 = jnp.dot(r1, w2s[...], preferred_element_type=jnp.float32) + pv[1:2, :]
    zm = jnp.dot(rm, w2[...], preferred_element_type=jnp.float32) + pv[5:6, :]
    out[:, 0:D] = z0 + zm
    out[:, D:2 * D] = z1 + zm


def _node_spec(off):
    return pl.BlockSpec((BT, D), lambda i, o=off: (i + o, 0))


def _full_spec(r):
    return pl.BlockSpec((r, D), lambda i: (0, 0))


def kernel(x, edge_index, eps1, W1s, b1s, g1s, be1s, W2s, b2s,
           eps2, W1, b1, g1, be1, W2, b2):
    src = edge_index[0].astype(jnp.int32)
    dst = edge_index[1].astype(jnp.int32)
    xs = jnp.concatenate([x[:, :D], x[:, D:]], axis=0)

    srcr = src.reshape(NS, NCHUNK, CHUNK)
    sidx = jnp.stack([srcr, srcr + N])
    didx = dst.reshape(NS, NCHUNK, CHUNK)

    gs = _sc_segsum()(xs, sidx, didx)

    pv = jnp.stack([
        b1s, b2s, g1s, be1s, b1, b2, g1, be1,
        jnp.full((D,), eps1, dtype=jnp.float32),
        jnp.full((D,), eps2, dtype=jnp.float32),
        jnp.zeros((D,), jnp.float32), jnp.zeros((D,), jnp.float32),
        jnp.zeros((D,), jnp.float32), jnp.zeros((D,), jnp.float32),
        jnp.zeros((D,), jnp.float32), jnp.zeros((D,), jnp.float32),
    ])

    stats = pl.pallas_call(
        _stats_body,
        grid=(NB,),
        in_specs=[
            _node_spec(0), _node_spec(NB), _node_spec(0), _node_spec(NB),
            _full_spec(D), _full_spec(D), _full_spec(16),
        ],
        out_specs=pl.BlockSpec((8, D), lambda i: (0, 0)),
        out_shape=jax.ShapeDtypeStruct((8, D), jnp.float32),
    )(xs, xs, gs, gs, W1s, W1, pv)

    out = pl.pallas_call(
        _apply_body,
        grid=(NB,),
        in_specs=[
            _node_spec(0), _node_spec(NB), _node_spec(0), _node_spec(NB),
            _full_spec(8),
            _full_spec(D), _full_spec(D), _full_spec(D), _full_spec(D),
            _full_spec(16),
        ],
        out_specs=pl.BlockSpec((BT, S * D), lambda i: (i, 0)),
        out_shape=jax.ShapeDtypeStruct((N, S * D), jnp.float32),
    )(xs, xs, gs, gs, stats, W1s, W2s, W1, W2, pv)

    return out


# speedup vs baseline: 1.1173x; 1.1173x over previous
import functools

import jax
import jax.numpy as jnp
from jax import lax
from jax.experimental import pallas as pl
from jax.experimental.pallas import tpu as pltpu
from jax.experimental.pallas import tpu_sc as plsc

N = 10000
E = 160000
D = 128
S = 2
NS = 16
ROWS_MAIN = 624
ROWS_TAIL = N - NS * ROWS_MAIN
EDGES_PER_TILE = E // NS
CHUNK = 80
NCHUNK = EDGES_PER_TILE // CHUNK
NBUF = 3
ROUNDS = NCHUNK // NBUF

BT = 1000
NB = N // BT


def _sc_body(x2_hbm, x_hbm, sidx_hbm, didx_hbm, out_hbm,
             sidx_v, didx_v, acc_sh,
             r0, r1, r2, g0, g1, g2, s0, s1, s2, d0, d1, d2):
    c = lax.axis_index("c")
    s = lax.axis_index("s")
    rows = (r0, r1, r2)
    gsem = (g0, g1, g2)
    ssem = (s0, s1, s2)
    dsem = (d0, d1, d2)
    row0 = s * ROWS_MAIN
    col0 = pl.multiple_of(c * D, D)

    pltpu.sync_copy(sidx_hbm.at[c, s], sidx_v)

    pltpu.sync_copy(x_hbm.at[pl.ds(row0, ROWS_MAIN), pl.ds(col0, D)],
                    acc_sh.at[pl.ds(row0, ROWS_MAIN)])

    @pl.when(s == 0)
    def _():
        pltpu.sync_copy(x_hbm.at[pl.ds(NS * ROWS_MAIN, ROWS_TAIL), pl.ds(col0, D)],
                        acc_sh.at[pl.ds(NS * ROWS_MAIN, ROWS_TAIL)])

    plsc.subcore_barrier()

    def start_gather(j, b):
        off = pl.multiple_of(j * CHUNK, CHUNK)
        pltpu.async_copy(x2_hbm.at[sidx_v.at[pl.ds(off, CHUNK)]], rows[b], gsem[b])

    def wait_gather(b):
        pltpu.make_async_copy(x2_hbm.at[sidx_v.at[pl.ds(0, CHUNK)]], rows[b],
                              gsem[b]).wait()

    def start_didx(j, b):
        pltpu.async_copy(didx_hbm.at[s, pl.ds(j, 1)], didx_v.at[pl.ds(b, 1)],
                         dsem[b])

    def wait_didx(b):
        pltpu.make_async_copy(didx_hbm.at[s, pl.ds(0, 1)],
                              didx_v.at[pl.ds(b, 1)], dsem[b]).wait()

    def start_scatter(b):
        pltpu.async_copy(rows[b], acc_sh.at[didx_v.at[b]], ssem[b], add=True)

    def wait_scatter(b):
        pltpu.make_async_copy(rows[b], acc_sh.at[didx_v.at[0]], ssem[b]).wait()

    for b in range(NBUF):
        start_gather(b, b)
        start_didx(b, b)

    def round_body(t, carry):
        j0 = t * NBUF
        for b in range(NBUF):
            wait_gather(b)
            wait_didx(b)
            start_scatter(b)
        for b in range(NBUF):
            wait_scatter(b)
            start_gather(j0 + NBUF + b, b)
            start_didx(j0 + NBUF + b, b)
        return carry

    lax.fori_loop(0, ROUNDS - 1, round_body, 0)

    for b in range(NBUF):
        wait_gather(b)
        wait_didx(b)
        start_scatter(b)
    for b in range(NBUF):
        wait_scatter(b)

    for j in range(ROUNDS * NBUF, NCHUNK):
        start_gather(j, 0)
        start_didx(j, 0)
        wait_gather(0)
        wait_didx(0)
        start_scatter(0)
        wait_scatter(0)

    plsc.subcore_barrier()
    pltpu.sync_copy(acc_sh.at[pl.ds(row0, ROWS_MAIN)],
                    out_hbm.at[pl.ds(c * N + row0, ROWS_MAIN)])

    @pl.when(s == 0)
    def _():
        pltpu.sync_copy(acc_sh.at[pl.ds(NS * ROWS_MAIN, ROWS_TAIL)],
                        out_hbm.at[pl.ds(c * N + NS * ROWS_MAIN, ROWS_TAIL)])


@functools.cache
def _sc_segsum():
    return pl.kernel(
        _sc_body,
        mesh=plsc.VectorSubcoreMesh(core_axis_name="c", subcore_axis_name="s"),
        out_type=jax.ShapeDtypeStruct((S * N, D), jnp.float32),
        scratch_types=[
            pltpu.VMEM((EDGES_PER_TILE,), jnp.int32),
            pltpu.VMEM((NBUF, CHUNK), jnp.int32),
            pltpu.VMEM_SHARED((N, D), jnp.float32),
        ] + [pltpu.VMEM((CHUNK, D), jnp.float32)] * NBUF
          + [pltpu.SemaphoreType.DMA] * (3 * NBUF),
    )


def _stats_body(xb, g0, g1, w1s, w1, pv, out):
    i = pl.program_id(0)

    @pl.when(i == 0)
    def _():
        out[...] = jnp.zeros_like(out)

    x0 = xb[:, 0:D]
    x1 = xb[:, D:2 * D]
    e1 = pv[8:9, :]
    e2 = pv[9:10, :]
    h0 = g0[...] + e1 * x0
    h1 = g1[...] + e1 * x1
    hm = 0.5 * (g0[...] + g1[...]) + 0.5 * e2 * (x0 + x1)
    a0 = jnp.dot(h0, w1s[...], preferred_element_type=jnp.float32) + pv[0:1, :]
    a1 = jnp.dot(h1, w1s[...], preferred_element_type=jnp.float32) + pv[0:1, :]
    am = jnp.dot(hm, w1[...], preferred_element_type=jnp.float32) + pv[4:5, :]
    s_s = jnp.sum(a0, axis=0) + jnp.sum(a1, axis=0)
    q_s = jnp.sum(a0 * a0, axis=0) + jnp.sum(a1 * a1, axis=0)
    s_a = jnp.sum(am, axis=0)
    q_a = jnp.sum(am * am, axis=0)
    z = jnp.zeros_like(s_s)
    upd = jnp.stack([s_s, q_s, s_a, q_a, z, z, z, z])
    out[...] = out[...] + upd


def _apply_body(xb, g0, g1, st, w1s, w2s, w1, w2, pv, out):
    x0 = xb[:, 0:D]
    x1 = xb[:, D:2 * D]
    e1 = pv[8:9, :]
    e2 = pv[9:10, :]
    h0 = g0[...] + e1 * x0
    h1 = g1[...] + e1 * x1
    hm = 0.5 * (g0[...] + g1[...]) + 0.5 * e2 * (x0 + x1)

    inv_ms = 1.0 / float(S * N)
    inv_ma = 1.0 / float(N)
    mean_s = st[0:1, :] * inv_ms
    var_s = st[1:2, :] * inv_ms - mean_s * mean_s
    scale_s = pv[2:3, :] * lax.rsqrt(var_s + 1e-5)
    shift_s = pv[3:4, :] - mean_s * scale_s
    mean_a = st[2:3, :] * inv_ma
    var_a = st[3:4, :] * inv_ma - mean_a * mean_a
    scale_a = pv[6:7, :] * lax.rsqrt(var_a + 1e-5)
    shift_a = pv[7:8, :] - mean_a * scale_a

    a0 = jnp.dot(h0, w1s[...], preferred_element_type=jnp.float32) + pv[0:1, :]
    a1 = jnp.dot(h1, w1s[...], preferred_element_type=jnp.float32) + pv[0:1, :]
    am = jnp.dot(hm, w1[...], preferred_element_type=jnp.float32) + pv[4:5, :]
    r0 = jnp.maximum(a0 * scale_s + shift_s, 0.0)
    r1 = jnp.maximum(a1 * scale_s + shift_s, 0.0)
    rm = jnp.maximum(am * scale_a + shift_a, 0.0)
    z0 = jnp.dot(r0, w2s[...], preferred_element_type=jnp.float32) + pv[1:2, :]
    z1 = jnp.dot(r1, w2s[...], preferred_element_type=jnp.float32) + pv[1:2, :]
    zm = jnp.dot(rm, w2[...], preferred_element_type=jnp.float32) + pv[5:6, :]
    out[:, 0:D] = z0 + zm
    out[:, D:2 * D] = z1 + zm


def _node_spec(off):
    return pl.BlockSpec((BT, D), lambda i, o=off: (i + o, 0))


def _full_spec(r):
    return pl.BlockSpec((r, D), lambda i: (0, 0))


def kernel(x, edge_index, eps1, W1s, b1s, g1s, be1s, W2s, b2s,
           eps2, W1, b1, g1, be1, W2, b2):
    src = edge_index[0].astype(jnp.int32)
    dst = edge_index[1].astype(jnp.int32)
    x2 = x.reshape(S * N, D)

    srcr = 2 * src.reshape(NS, EDGES_PER_TILE)
    sidx = jnp.stack([srcr, srcr + 1])
    didx = dst.reshape(NS, NCHUNK, CHUNK)

    gs = _sc_segsum()(x2, x, sidx, didx)

    pv = jnp.stack([
        b1s, b2s, g1s, be1s, b1, b2, g1, be1,
        jnp.full((D,), eps1, dtype=jnp.float32),
        jnp.full((D,), eps2, dtype=jnp.float32),
        jnp.zeros((D,), jnp.float32), jnp.zeros((D,), jnp.float32),
        jnp.zeros((D,), jnp.float32), jnp.zeros((D,), jnp.float32),
        jnp.zeros((D,), jnp.float32), jnp.zeros((D,), jnp.float32),
    ])

    xspec = pl.BlockSpec((BT, S * D), lambda i: (i, 0))

    stats = pl.pallas_call(
        _stats_body,
        grid=(NB,),
        in_specs=[
            xspec, _node_spec(0), _node_spec(NB),
            _full_spec(D), _full_spec(D), _full_spec(16),
        ],
        out_specs=pl.BlockSpec((8, D), lambda i: (0, 0)),
        out_shape=jax.ShapeDtypeStruct((8, D), jnp.float32),
    )(x, gs, gs, W1s, W1, pv)

    out = pl.pallas_call(
        _apply_body,
        grid=(NB,),
        in_specs=[
            xspec, _node_spec(0), _node_spec(NB),
            _full_spec(8),
            _full_spec(D), _full_spec(D), _full_spec(D), _full_spec(D),
            _full_spec(16),
        ],
        out_specs=pl.BlockSpec((BT, S * D), lambda i: (i, 0)),
        out_shape=jax.ShapeDtypeStruct((N, S * D), jnp.float32),
    )(x, gs, gs, stats, W1s, W2s, W1, W2, pv)

    return out


# speedup vs baseline: 1.1659x; 1.0435x over previous
import functools

import jax
import jax.numpy as jnp
from jax import lax
from jax.experimental import pallas as pl
from jax.experimental.pallas import tpu as pltpu
from jax.experimental.pallas import tpu_sc as plsc

N = 10000
E = 160000
D = 128
S = 2
NS = 16
ROWS_MAIN = 624
ROWS_TAIL = N - NS * ROWS_MAIN
EDGES_PER_TILE = E // NS
CHUNK = 80
NCHUNK = EDGES_PER_TILE // CHUNK
NBUF = 3
ROUNDS = NCHUNK // NBUF

BT = 1000
NB = N // BT


def _sc_body(x2_hbm, x_hbm, sidx_hbm, didx_hbm, out_hbm,
             sidx_v, didx_v, acc_sh,
             r0, r1, r2, g0, g1, g2, s0, s1, s2, d0, d1, d2):
    c = lax.axis_index("c")
    s = lax.axis_index("s")
    rows = (r0, r1, r2)
    gsem = (g0, g1, g2)
    ssem = (s0, s1, s2)
    dsem = (d0, d1, d2)
    row0 = s * ROWS_MAIN
    col0 = pl.multiple_of(c * D, D)

    pltpu.sync_copy(sidx_hbm.at[c, s], sidx_v)

    pltpu.sync_copy(x_hbm.at[pl.ds(row0, ROWS_MAIN), pl.ds(col0, D)],
                    acc_sh.at[pl.ds(row0, ROWS_MAIN)])

    @pl.when(s == 0)
    def _():
        pltpu.sync_copy(x_hbm.at[pl.ds(NS * ROWS_MAIN, ROWS_TAIL), pl.ds(col0, D)],
                        acc_sh.at[pl.ds(NS * ROWS_MAIN, ROWS_TAIL)])

    plsc.subcore_barrier()

    def start_gather(j, b):
        off = pl.multiple_of(j * CHUNK, CHUNK)
        pltpu.async_copy(x2_hbm.at[sidx_v.at[pl.ds(off, CHUNK)]], rows[b], gsem[b])

    def wait_gather(b):
        pltpu.make_async_copy(x2_hbm.at[sidx_v.at[pl.ds(0, CHUNK)]], rows[b],
                              gsem[b]).wait()

    def start_didx(j, b):
        pltpu.async_copy(didx_hbm.at[s, pl.ds(j, 1)], didx_v.at[pl.ds(b, 1)],
                         dsem[b])

    def wait_didx(b):
        pltpu.make_async_copy(didx_hbm.at[s, pl.ds(0, 1)],
                              didx_v.at[pl.ds(b, 1)], dsem[b]).wait()

    def start_scatter(b):
        pltpu.async_copy(rows[b], acc_sh.at[didx_v.at[b]], ssem[b], add=True)

    def wait_scatter(b):
        pltpu.make_async_copy(rows[b], acc_sh.at[didx_v.at[0]], ssem[b]).wait()

    for b in range(NBUF):
        start_gather(b, b)
        start_didx(b, b)

    def round_body(t, carry):
        j0 = t * NBUF
        for b in range(NBUF):
            wait_gather(b)
            wait_didx(b)
            start_scatter(b)
        for b in range(NBUF):
            wait_scatter(b)
            start_gather(j0 + NBUF + b, b)
            start_didx(j0 + NBUF + b, b)
        return carry

    lax.fori_loop(0, ROUNDS - 1, round_body, 0)

    for b in range(NBUF):
        wait_gather(b)
        wait_didx(b)
        start_scatter(b)
    for b in range(NBUF):
        wait_scatter(b)

    for j in range(ROUNDS * NBUF, NCHUNK):
        start_gather(j, 0)
        start_didx(j, 0)
        wait_gather(0)
        wait_didx(0)
        start_scatter(0)
        wait_scatter(0)

    plsc.subcore_barrier()
    pltpu.sync_copy(acc_sh.at[pl.ds(row0, ROWS_MAIN)],
                    out_hbm.at[pl.ds(c * N + row0, ROWS_MAIN)])

    @pl.when(s == 0)
    def _():
        pltpu.sync_copy(acc_sh.at[pl.ds(NS * ROWS_MAIN, ROWS_TAIL)],
                        out_hbm.at[pl.ds(c * N + NS * ROWS_MAIN, ROWS_TAIL)])


@functools.cache
def _sc_segsum():
    return pl.kernel(
        _sc_body,
        mesh=plsc.VectorSubcoreMesh(core_axis_name="c", subcore_axis_name="s"),
        out_type=jax.ShapeDtypeStruct((S * N, D), jnp.float32),
        scratch_types=[
            pltpu.VMEM((EDGES_PER_TILE,), jnp.int32),
            pltpu.VMEM((NBUF, CHUNK), jnp.int32),
            pltpu.VMEM_SHARED((N, D), jnp.float32),
        ] + [pltpu.VMEM((CHUNK, D), jnp.float32)] * NBUF
          + [pltpu.SemaphoreType.DMA] * (3 * NBUF),
    )


def _mlp_body(xb, g0, g1, w1s, w2s, w1, w2, pv, out, a0s, a1s, ams, st):
    p = pl.program_id(0)
    i = pl.program_id(1)
    off = pl.multiple_of(i * BT, BT)

    @pl.when((p == 0) & (i == 0))
    def _():
        st[...] = jnp.zeros_like(st)

    @pl.when(p == 0)
    def _():
        x0 = xb[:, 0:D]
        x1 = xb[:, D:2 * D]
        e1 = pv[8:9, :]
        e2 = pv[9:10, :]
        h0 = g0[...] + e1 * x0
        h1 = g1[...] + e1 * x1
        hm = 0.5 * (g0[...] + g1[...]) + 0.5 * e2 * (x0 + x1)
        a0 = jnp.dot(h0, w1s[...], preferred_element_type=jnp.float32) + pv[0:1, :]
        a1 = jnp.dot(h1, w1s[...], preferred_element_type=jnp.float32) + pv[0:1, :]
        am = jnp.dot(hm, w1[...], preferred_element_type=jnp.float32) + pv[4:5, :]
        a0s[pl.ds(off, BT), :] = a0
        a1s[pl.ds(off, BT), :] = a1
        ams[pl.ds(off, BT), :] = am
        s_s = jnp.sum(a0, axis=0) + jnp.sum(a1, axis=0)
        q_s = jnp.sum(a0 * a0, axis=0) + jnp.sum(a1 * a1, axis=0)
        s_a = jnp.sum(am, axis=0)
        q_a = jnp.sum(am * am, axis=0)
        z = jnp.zeros_like(s_s)
        st[...] = st[...] + jnp.stack([s_s, q_s, s_a, q_a, z, z, z, z])

    @pl.when(p == 1)
    def _():
        inv_ms = 1.0 / float(S * N)
        inv_ma = 1.0 / float(N)
        mean_s = st[0:1, :] * inv_ms
        var_s = st[1:2, :] * inv_ms - mean_s * mean_s
        scale_s = pv[2:3, :] * lax.rsqrt(var_s + 1e-5)
        shift_s = pv[3:4, :] - mean_s * scale_s
        mean_a = st[2:3, :] * inv_ma
        var_a = st[3:4, :] * inv_ma - mean_a * mean_a
        scale_a = pv[6:7, :] * lax.rsqrt(var_a + 1e-5)
        shift_a = pv[7:8, :] - mean_a * scale_a

        a0 = a0s[pl.ds(off, BT), :]
        a1 = a1s[pl.ds(off, BT), :]
        am = ams[pl.ds(off, BT), :]
        r0 = jnp.maximum(a0 * scale_s + shift_s, 0.0)
        r1 = jnp.maximum(a1 * scale_s + shift_s, 0.0)
        rm = jnp.maximum(am * scale_a + shift_a, 0.0)
        z0 = jnp.dot(r0, w2s[...], preferred_element_type=jnp.float32) + pv[1:2, :]
        z1 = jnp.dot(r1, w2s[...], preferred_element_type=jnp.float32) + pv[1:2, :]
        zm = jnp.dot(rm, w2[...], preferred_element_type=jnp.float32) + pv[5:6, :]
        out[:, 0:D] = z0 + zm
        out[:, D:2 * D] = z1 + zm


def _node_spec(off):
    return pl.BlockSpec((BT, D), lambda i, o=off: (i + o, 0))


def _full_spec(r):
    return pl.BlockSpec((r, D), lambda i: (0, 0))


def kernel(x, edge_index, eps1, W1s, b1s, g1s, be1s, W2s, b2s,
           eps2, W1, b1, g1, be1, W2, b2):
    src = edge_index[0].astype(jnp.int32)
    dst = edge_index[1].astype(jnp.int32)
    x2 = x.reshape(S * N, D)

    srcr = 2 * src.reshape(NS, EDGES_PER_TILE)
    sidx = jnp.stack([srcr, srcr + 1])
    didx = dst.reshape(NS, NCHUNK, CHUNK)

    gs = _sc_segsum()(x2, x, sidx, didx)

    pv = jnp.stack([
        b1s, b2s, g1s, be1s, b1, b2, g1, be1,
        jnp.full((D,), eps1, dtype=jnp.float32),
        jnp.full((D,), eps2, dtype=jnp.float32),
        jnp.zeros((D,), jnp.float32), jnp.zeros((D,), jnp.float32),
        jnp.zeros((D,), jnp.float32), jnp.zeros((D,), jnp.float32),
        jnp.zeros((D,), jnp.float32), jnp.zeros((D,), jnp.float32),
    ])

    def p0_map(p, i):
        return (jnp.where(p == 0, i, 0), 0)

    def p0_map_off(o):
        return lambda p, i: (jnp.where(p == 0, i + o, o), 0)

    out = pl.pallas_call(
        _mlp_body,
        grid=(2, NB),
        in_specs=[
            pl.BlockSpec((BT, S * D), p0_map),
            pl.BlockSpec((BT, D), p0_map_off(0)),
            pl.BlockSpec((BT, D), p0_map_off(NB)),
            pl.BlockSpec((D, D), lambda p, i: (0, 0)),
            pl.BlockSpec((D, D), lambda p, i: (0, 0)),
            pl.BlockSpec((D, D), lambda p, i: (0, 0)),
            pl.BlockSpec((D, D), lambda p, i: (0, 0)),
            pl.BlockSpec((16, D), lambda p, i: (0, 0)),
        ],
        out_specs=pl.BlockSpec((BT, S * D), lambda p, i: (jnp.where(p == 1, i, 0), 0)),
        out_shape=jax.ShapeDtypeStruct((N, S * D), jnp.float32),
        scratch_shapes=[
            pltpu.VMEM((N, D), jnp.float32),
            pltpu.VMEM((N, D), jnp.float32),
            pltpu.VMEM((N, D), jnp.float32),
            pltpu.VMEM((8, D), jnp.float32),
        ],
    )(x, gs, gs, W1s, W2s, W1, W2, pv)

    return out


# speedup vs baseline: 1.1849x; 1.0163x over previous
import functools

import jax
import jax.numpy as jnp
from jax import lax
from jax.experimental import pallas as pl
from jax.experimental.pallas import tpu as pltpu
from jax.experimental.pallas import tpu_sc as plsc

N = 10000
E = 160000
D = 128
S = 2
NS = 16
ROWS_MAIN = 624
ROWS_TAIL = N - NS * ROWS_MAIN
EDGES_PER_TILE = E // NS
CHUNK = 80
NCHUNK = EDGES_PER_TILE // CHUNK
NBUF = 3
ROUNDS = NCHUNK // NBUF

BT = 1000
NB = N // BT


def _sc_body(x2_hbm, x_hbm, sidx_hbm, didx_hbm, out_hbm,
             sidx_v, didx_v, acc_sh,
             r0, r1, r2, g0, g1, g2, s0, s1, s2, d0, d1, d2, isem):
    c = lax.axis_index("c")
    s = lax.axis_index("s")
    rows = (r0, r1, r2)
    gsem = (g0, g1, g2)
    ssem = (s0, s1, s2)
    dsem = (d0, d1, d2)
    row0 = s * ROWS_MAIN
    col0 = pl.multiple_of(c * D, D)

    init_main = pltpu.async_copy(
        x_hbm.at[pl.ds(row0, ROWS_MAIN), pl.ds(col0, D)],
        acc_sh.at[pl.ds(row0, ROWS_MAIN)], isem)

    @pl.when(s == 0)
    def _():
        pltpu.async_copy(
            x_hbm.at[pl.ds(NS * ROWS_MAIN, ROWS_TAIL), pl.ds(col0, D)],
            acc_sh.at[pl.ds(NS * ROWS_MAIN, ROWS_TAIL)], isem)

    pltpu.sync_copy(sidx_hbm.at[c, s], sidx_v)

    def start_gather(j, b):
        off = pl.multiple_of(j * CHUNK, CHUNK)
        pltpu.async_copy(x2_hbm.at[sidx_v.at[pl.ds(off, CHUNK)]], rows[b], gsem[b])

    def wait_gather(b):
        pltpu.make_async_copy(x2_hbm.at[sidx_v.at[pl.ds(0, CHUNK)]], rows[b],
                              gsem[b]).wait()

    def start_didx(j, b):
        pltpu.async_copy(didx_hbm.at[s, pl.ds(j, 1)], didx_v.at[pl.ds(b, 1)],
                         dsem[b])

    def wait_didx(b):
        pltpu.make_async_copy(didx_hbm.at[s, pl.ds(0, 1)],
                              didx_v.at[pl.ds(b, 1)], dsem[b]).wait()

    def start_scatter(b):
        pltpu.async_copy(rows[b], acc_sh.at[didx_v.at[b]], ssem[b], add=True)

    def wait_scatter(b):
        pltpu.make_async_copy(rows[b], acc_sh.at[didx_v.at[0]], ssem[b]).wait()

    for b in range(NBUF):
        start_gather(b, b)
        start_didx(b, b)

    init_main.wait()

    @pl.when(s == 0)
    def _():
        pltpu.make_async_copy(
            x_hbm.at[pl.ds(NS * ROWS_MAIN, ROWS_TAIL), pl.ds(col0, D)],
            acc_sh.at[pl.ds(NS * ROWS_MAIN, ROWS_TAIL)], isem).wait()

    plsc.subcore_barrier()

    def round_body(t, carry):
        j0 = t * NBUF
        for b in range(NBUF):
            wait_gather(b)
            wait_didx(b)
            start_scatter(b)
        for b in range(NBUF):
            wait_scatter(b)
            start_gather(j0 + NBUF + b, b)
            start_didx(j0 + NBUF + b, b)
        return carry

    lax.fori_loop(0, ROUNDS - 1, round_body, 0)

    for b in range(NBUF):
        wait_gather(b)
        wait_didx(b)
        start_scatter(b)
    for b in range(NBUF):
        wait_scatter(b)

    for j in range(ROUNDS * NBUF, NCHUNK):
        start_gather(j, 0)
        start_didx(j, 0)
        wait_gather(0)
        wait_didx(0)
        start_scatter(0)
        wait_scatter(0)

    plsc.subcore_barrier()
    pltpu.sync_copy(acc_sh.at[pl.ds(row0, ROWS_MAIN)],
                    out_hbm.at[pl.ds(c * N + row0, ROWS_MAIN)])

    @pl.when(s == 0)
    def _():
        pltpu.sync_copy(acc_sh.at[pl.ds(NS * ROWS_MAIN, ROWS_TAIL)],
                        out_hbm.at[pl.ds(c * N + NS * ROWS_MAIN, ROWS_TAIL)])


@functools.cache
def _sc_segsum():
    return pl.kernel(
        _sc_body,
        mesh=plsc.VectorSubcoreMesh(core_axis_name="c", subcore_axis_name="s"),
        out_type=jax.ShapeDtypeStruct((S * N, D), jnp.float32),
        scratch_types=[
            pltpu.VMEM((EDGES_PER_TILE,), jnp.int32),
            pltpu.VMEM((NBUF, CHUNK), jnp.int32),
            pltpu.VMEM_SHARED((N, D), jnp.float32),
        ] + [pltpu.VMEM((CHUNK, D), jnp.float32)] * NBUF
          + [pltpu.SemaphoreType.DMA] * (3 * NBUF + 1),
    )


def _mlp_body(xb, g0, g1, w1s, w2s, w1, w2, pv, out, a0s, a1s, ams, st):
    p = pl.program_id(0)
    i = pl.program_id(1)
    off = pl.multiple_of(i * BT, BT)

    @pl.when((p == 0) & (i == 0))
    def _():
        st[...] = jnp.zeros_like(st)

    @pl.when(p == 0)
    def _():
        x0 = xb[:, 0:D]
        x1 = xb[:, D:2 * D]
        e1 = pv[8:9, :]
        e2 = pv[9:10, :]
        h0 = g0[...] + e1 * x0
        h1 = g1[...] + e1 * x1
        hm = 0.5 * (g0[...] + g1[...]) + 0.5 * e2 * (x0 + x1)
        a0 = jnp.dot(h0, w1s[...], preferred_element_type=jnp.float32) + pv[0:1, :]
        a1 = jnp.dot(h1, w1s[...], preferred_element_type=jnp.float32) + pv[0:1, :]
        am = jnp.dot(hm, w1[...], preferred_element_type=jnp.float32) + pv[4:5, :]
        a0s[pl.ds(off, BT), :] = a0
        a1s[pl.ds(off, BT), :] = a1
        ams[pl.ds(off, BT), :] = am
        s_s = jnp.sum(a0, axis=0) + jnp.sum(a1, axis=0)
        q_s = jnp.sum(a0 * a0, axis=0) + jnp.sum(a1 * a1, axis=0)
        s_a = jnp.sum(am, axis=0)
        q_a = jnp.sum(am * am, axis=0)
        z = jnp.zeros_like(s_s)
        st[...] = st[...] + jnp.stack([s_s, q_s, s_a, q_a, z, z, z, z])

    @pl.when(p == 1)
    def _():
        inv_ms = 1.0 / float(S * N)
        inv_ma = 1.0 / float(N)
        mean_s = st[0:1, :] * inv_ms
        var_s = st[1:2, :] * inv_ms - mean_s * mean_s
        scale_s = pv[2:3, :] * lax.rsqrt(var_s + 1e-5)
        shift_s = pv[3:4, :] - mean_s * scale_s
        mean_a = st[2:3, :] * inv_ma
        var_a = st[3:4, :] * inv_ma - mean_a * mean_a
        scale_a = pv[6:7, :] * lax.rsqrt(var_a + 1e-5)
        shift_a = pv[7:8, :] - mean_a * scale_a

        a0 = a0s[pl.ds(off, BT), :]
        a1 = a1s[pl.ds(off, BT), :]
        am = ams[pl.ds(off, BT), :]
        r0 = jnp.maximum(a0 * scale_s + shift_s, 0.0)
        r1 = jnp.maximum(a1 * scale_s + shift_s, 0.0)
        rm = jnp.maximum(am * scale_a + shift_a, 0.0)
        z0 = jnp.dot(r0, w2s[...], preferred_element_type=jnp.float32) + pv[1:2, :]
        z1 = jnp.dot(r1, w2s[...], preferred_element_type=jnp.float32) + pv[1:2, :]
        zm = jnp.dot(rm, w2[...], preferred_element_type=jnp.float32) + pv[5:6, :]
        out[:, 0:D] = z0 + zm
        out[:, D:2 * D] = z1 + zm


def _node_spec(off):
    return pl.BlockSpec((BT, D), lambda i, o=off: (i + o, 0))


def _full_spec(r):
    return pl.BlockSpec((r, D), lambda i: (0, 0))


def kernel(x, edge_index, eps1, W1s, b1s, g1s, be1s, W2s, b2s,
           eps2, W1, b1, g1, be1, W2, b2):
    src = edge_index[0].astype(jnp.int32)
    dst = edge_index[1].astype(jnp.int32)
    x2 = x.reshape(S * N, D)

    srcr = 2 * src.reshape(NS, EDGES_PER_TILE)
    sidx = jnp.stack([srcr, srcr + 1])
    didx = dst.reshape(NS, NCHUNK, CHUNK)

    gs = _sc_segsum()(x2, x, sidx, didx)

    pv = jnp.stack([
        b1s, b2s, g1s, be1s, b1, b2, g1, be1,
        jnp.full((D,), eps1, dtype=jnp.float32),
        jnp.full((D,), eps2, dtype=jnp.float32),
        jnp.zeros((D,), jnp.float32), jnp.zeros((D,), jnp.float32),
        jnp.zeros((D,), jnp.float32), jnp.zeros((D,), jnp.float32),
        jnp.zeros((D,), jnp.float32), jnp.zeros((D,), jnp.float32),
    ])

    def p0_map(p, i):
        return (jnp.where(p == 0, i, 0), 0)

    def p0_map_off(o):
        return lambda p, i: (jnp.where(p == 0, i + o, o), 0)

    out = pl.pallas_call(
        _mlp_body,
        grid=(2, NB),
        in_specs=[
            pl.BlockSpec((BT, S * D), p0_map),
            pl.BlockSpec((BT, D), p0_map_off(0)),
            pl.BlockSpec((BT, D), p0_map_off(NB)),
            pl.BlockSpec((D, D), lambda p, i: (0, 0)),
            pl.BlockSpec((D, D), lambda p, i: (0, 0)),
            pl.BlockSpec((D, D), lambda p, i: (0, 0)),
            pl.BlockSpec((D, D), lambda p, i: (0, 0)),
            pl.BlockSpec((16, D), lambda p, i: (0, 0)),
        ],
        out_specs=pl.BlockSpec((BT, S * D), lambda p, i: (jnp.where(p == 1, i, 0), 0)),
        out_shape=jax.ShapeDtypeStruct((N, S * D), jnp.float32),
        scratch_shapes=[
            pltpu.VMEM((N, D), jnp.float32),
            pltpu.VMEM((N, D), jnp.float32),
            pltpu.VMEM((N, D), jnp.float32),
            pltpu.VMEM((8, D), jnp.float32),
        ],
    )(x, gs, gs, W1s, W2s, W1, W2, pv)

    return out


# speedup vs baseline: 1.2261x; 1.0348x over previous
import functools

import jax
import jax.numpy as jnp
from jax import lax
from jax.experimental import pallas as pl
from jax.experimental.pallas import tpu as pltpu
from jax.experimental.pallas import tpu_sc as plsc

N = 10000
E = 160000
D = 128
S = 2
NS = 16
ROWS_MAIN = 624
ROWS_TAIL = N - NS * ROWS_MAIN
EDGES_PER_TILE = E // NS
CHUNK = 80
NCHUNK = EDGES_PER_TILE // CHUNK
NBUF = 3
ROUNDS = NCHUNK // NBUF

BT = 2000
NB = N // BT


def _sc_body(x2_hbm, x_hbm, sidx_hbm, didx_hbm, out_hbm,
             sidx_v, didx_v, acc_sh,
             r0, r1, r2, g0, g1, g2, s0, s1, s2, d0, d1, d2, isem):
    c = lax.axis_index("c")
    s = lax.axis_index("s")
    rows = (r0, r1, r2)
    gsem = (g0, g1, g2)
    ssem = (s0, s1, s2)
    dsem = (d0, d1, d2)
    row0 = s * ROWS_MAIN
    col0 = pl.multiple_of(c * D, D)

    init_main = pltpu.async_copy(
        x_hbm.at[pl.ds(row0, ROWS_MAIN), pl.ds(col0, D)],
        acc_sh.at[pl.ds(row0, ROWS_MAIN)], isem)

    @pl.when(s == 0)
    def _():
        pltpu.async_copy(
            x_hbm.at[pl.ds(NS * ROWS_MAIN, ROWS_TAIL), pl.ds(col0, D)],
            acc_sh.at[pl.ds(NS * ROWS_MAIN, ROWS_TAIL)], isem)

    pltpu.sync_copy(sidx_hbm.at[c, s], sidx_v)

    def start_gather(j, b):
        off = pl.multiple_of(j * CHUNK, CHUNK)
        pltpu.async_copy(x2_hbm.at[sidx_v.at[pl.ds(off, CHUNK)]], rows[b], gsem[b])

    def wait_gather(b):
        pltpu.make_async_copy(x2_hbm.at[sidx_v.at[pl.ds(0, CHUNK)]], rows[b],
                              gsem[b]).wait()

    def start_didx(j, b):
        pltpu.async_copy(didx_hbm.at[s, pl.ds(j, 1)], didx_v.at[pl.ds(b, 1)],
                         dsem[b])

    def wait_didx(b):
        pltpu.make_async_copy(didx_hbm.at[s, pl.ds(0, 1)],
                              didx_v.at[pl.ds(b, 1)], dsem[b]).wait()

    def start_scatter(b):
        pltpu.async_copy(rows[b], acc_sh.at[didx_v.at[b]], ssem[b], add=True)

    def wait_scatter(b):
        pltpu.make_async_copy(rows[b], acc_sh.at[didx_v.at[0]], ssem[b]).wait()

    for b in range(NBUF):
        start_gather(b, b)
        start_didx(b, b)

    init_main.wait()

    @pl.when(s == 0)
    def _():
        pltpu.make_async_copy(
            x_hbm.at[pl.ds(NS * ROWS_MAIN, ROWS_TAIL), pl.ds(col0, D)],
            acc_sh.at[pl.ds(NS * ROWS_MAIN, ROWS_TAIL)], isem).wait()

    plsc.subcore_barrier()

    def round_body(t, carry):
        j0 = t * NBUF
        for b in range(NBUF):
            wait_gather(b)
            wait_didx(b)
            start_scatter(b)
        for b in range(NBUF):
            wait_scatter(b)
            start_gather(j0 + NBUF + b, b)
            start_didx(j0 + NBUF + b, b)
        return carry

    lax.fori_loop(0, ROUNDS - 1, round_body, 0)

    for b in range(NBUF):
        wait_gather(b)
        wait_didx(b)
        start_scatter(b)
    for b in range(NBUF):
        wait_scatter(b)

    for j in range(ROUNDS * NBUF, NCHUNK):
        start_gather(j, 0)
        start_didx(j, 0)
        wait_gather(0)
        wait_didx(0)
        start_scatter(0)
        wait_scatter(0)

    plsc.subcore_barrier()
    pltpu.sync_copy(acc_sh.at[pl.ds(row0, ROWS_MAIN)],
                    out_hbm.at[pl.ds(c * N + row0, ROWS_MAIN)])

    @pl.when(s == 0)
    def _():
        pltpu.sync_copy(acc_sh.at[pl.ds(NS * ROWS_MAIN, ROWS_TAIL)],
                        out_hbm.at[pl.ds(c * N + NS * ROWS_MAIN, ROWS_TAIL)])


@functools.cache
def _sc_segsum():
    return pl.kernel(
        _sc_body,
        mesh=plsc.VectorSubcoreMesh(core_axis_name="c", subcore_axis_name="s"),
        out_type=jax.ShapeDtypeStruct((S * N, D), jnp.float32),
        scratch_types=[
            pltpu.VMEM((EDGES_PER_TILE,), jnp.int32),
            pltpu.VMEM((NBUF, CHUNK), jnp.int32),
            pltpu.VMEM_SHARED((N, D), jnp.float32),
        ] + [pltpu.VMEM((CHUNK, D), jnp.float32)] * NBUF
          + [pltpu.SemaphoreType.DMA] * (3 * NBUF + 1),
    )


def _mlp_body(xb, g0, g1, w1s, w2s, w1, w2, pv, out, a0s, a1s, ams, st):
    p = pl.program_id(0)
    i = pl.program_id(1)
    off = pl.multiple_of(i * BT, BT)

    @pl.when((p == 0) & (i == 0))
    def _():
        st[...] = jnp.zeros_like(st)

    @pl.when(p == 0)
    def _():
        x0 = xb[:, 0:D]
        x1 = xb[:, D:2 * D]
        e1 = pv[8:9, :]
        e2 = pv[9:10, :]
        h0 = g0[...] + e1 * x0
        h1 = g1[...] + e1 * x1
        hm = 0.5 * (g0[...] + g1[...]) + 0.5 * e2 * (x0 + x1)
        a0 = jnp.dot(h0, w1s[...], preferred_element_type=jnp.float32) + pv[0:1, :]
        a1 = jnp.dot(h1, w1s[...], preferred_element_type=jnp.float32) + pv[0:1, :]
        am = jnp.dot(hm, w1[...], preferred_element_type=jnp.float32) + pv[4:5, :]
        a0s[pl.ds(off, BT), :] = a0
        a1s[pl.ds(off, BT), :] = a1
        ams[pl.ds(off, BT), :] = am
        s_s = jnp.sum(a0, axis=0) + jnp.sum(a1, axis=0)
        q_s = jnp.sum(a0 * a0, axis=0) + jnp.sum(a1 * a1, axis=0)
        s_a = jnp.sum(am, axis=0)
        q_a = jnp.sum(am * am, axis=0)
        z = jnp.zeros_like(s_s)
        st[...] = st[...] + jnp.stack([s_s, q_s, s_a, q_a, z, z, z, z])

    @pl.when(p == 1)
    def _():
        inv_ms = 1.0 / float(S * N)
        inv_ma = 1.0 / float(N)
        mean_s = st[0:1, :] * inv_ms
        var_s = st[1:2, :] * inv_ms - mean_s * mean_s
        scale_s = pv[2:3, :] * lax.rsqrt(var_s + 1e-5)
        shift_s = pv[3:4, :] - mean_s * scale_s
        mean_a = st[2:3, :] * inv_ma
        var_a = st[3:4, :] * inv_ma - mean_a * mean_a
        scale_a = pv[6:7, :] * lax.rsqrt(var_a + 1e-5)
        shift_a = pv[7:8, :] - mean_a * scale_a

        a0 = a0s[pl.ds(off, BT), :]
        a1 = a1s[pl.ds(off, BT), :]
        am = ams[pl.ds(off, BT), :]
        r0 = jnp.maximum(a0 * scale_s + shift_s, 0.0)
        r1 = jnp.maximum(a1 * scale_s + shift_s, 0.0)
        rm = jnp.maximum(am * scale_a + shift_a, 0.0)
        z0 = jnp.dot(r0, w2s[...], preferred_element_type=jnp.float32) + pv[1:2, :]
        z1 = jnp.dot(r1, w2s[...], preferred_element_type=jnp.float32) + pv[1:2, :]
        zm = jnp.dot(rm, w2[...], preferred_element_type=jnp.float32) + pv[5:6, :]
        out[:, 0:D] = z0 + zm
        out[:, D:2 * D] = z1 + zm


def _node_spec(off):
    return pl.BlockSpec((BT, D), lambda i, o=off: (i + o, 0))


def _full_spec(r):
    return pl.BlockSpec((r, D), lambda i: (0, 0))


def kernel(x, edge_index, eps1, W1s, b1s, g1s, be1s, W2s, b2s,
           eps2, W1, b1, g1, be1, W2, b2):
    src = edge_index[0].astype(jnp.int32)
    dst = edge_index[1].astype(jnp.int32)
    x2 = x.reshape(S * N, D)

    srcr = 2 * src.reshape(NS, EDGES_PER_TILE)
    sidx = jnp.stack([srcr, srcr + 1])
    didx = dst.reshape(NS, NCHUNK, CHUNK)

    gs = _sc_segsum()(x2, x, sidx, didx)

    pv = jnp.stack([
        b1s, b2s, g1s, be1s, b1, b2, g1, be1,
        jnp.full((D,), eps1, dtype=jnp.float32),
        jnp.full((D,), eps2, dtype=jnp.float32),
        jnp.zeros((D,), jnp.float32), jnp.zeros((D,), jnp.float32),
        jnp.zeros((D,), jnp.float32), jnp.zeros((D,), jnp.float32),
        jnp.zeros((D,), jnp.float32), jnp.zeros((D,), jnp.float32),
    ])

    def p0_map(p, i):
        return (jnp.where(p == 0, i, 0), 0)

    def p0_map_off(o):
        return lambda p, i: (jnp.where(p == 0, i + o, o), 0)

    out = pl.pallas_call(
        _mlp_body,
        grid=(2, NB),
        in_specs=[
            pl.BlockSpec((BT, S * D), p0_map),
            pl.BlockSpec((BT, D), p0_map_off(0)),
            pl.BlockSpec((BT, D), p0_map_off(NB)),
            pl.BlockSpec((D, D), lambda p, i: (0, 0)),
            pl.BlockSpec((D, D), lambda p, i: (0, 0)),
            pl.BlockSpec((D, D), lambda p, i: (0, 0)),
            pl.BlockSpec((D, D), lambda p, i: (0, 0)),
            pl.BlockSpec((16, D), lambda p, i: (0, 0)),
        ],
        out_specs=pl.BlockSpec((BT, S * D), lambda p, i: (jnp.where(p == 1, i, 0), 0)),
        out_shape=jax.ShapeDtypeStruct((N, S * D), jnp.float32),
        scratch_shapes=[
            pltpu.VMEM((N, D), jnp.float32),
            pltpu.VMEM((N, D), jnp.float32),
            pltpu.VMEM((N, D), jnp.float32),
            pltpu.VMEM((8, D), jnp.float32),
        ],
    )(x, gs, gs, W1s, W2s, W1, W2, pv)

    return out
